# baseline (speedup 1.0000x reference)
import functools

import jax
import jax.numpy as jnp
from jax import lax
from jax.experimental import pallas as pl
from jax.experimental.pallas import tpu as pltpu
from jax.experimental.pallas import tpu_sc as plsc

THRESH = 1.0
GAP = 5
T = 512
L = 16
NC = 2
NS = 16
NW = NC * NS
ROWS = 32768
SPLIT = 16384
ROWS_PER_W = (ROWS - SPLIT) // NW
RB = 32
NBLK = ROWS_PER_W // RB
NCH = T // L
BIG = 1 << 22
RTC = 256

_GDN = lax.GatherDimensionNumbers(
    offset_dims=(), collapsed_slice_dims=(0,), start_index_map=(0,)
)


def _vgather(x, idx):
    return lax.gather(
        x,
        idx[:, None],
        _GDN,
        slice_sizes=(1,),
        mode=lax.GatherScatterMode.PROMISE_IN_BOUNDS,
    )


def _splat_last(x):
    return _vgather(x, jnp.full((L,), L - 1, jnp.int32))


def _sc_body(v_hbm, lab_hbm, out_hbm, rows_v, posbuf, labels_v, accbuf, sems):
    wid = lax.axis_index("s") * NC + lax.axis_index("c")
    wstart = SPLIT + wid * ROWS_PER_W
    iota = lax.iota(jnp.int32, L)
    lane0 = iota == 0
    shl = jnp.maximum(iota - 1, 0)

    pltpu.sync_copy(lab_hbm.at[pl.ds(wstart, ROWS_PER_W)], labels_v)

    def row_body(r, rbase, d, acc):
        g = rbase + r
        labv = labels_v[pl.ds((g // L) * L, L)]
        lab_vec = _vgather(labv, jnp.full((L,), g % L, jnp.int32))
        lab_s = lab_vec[0]

        def miss_branch(_):
            @plsc.parallel_loop(
                0, NCH, carry=jnp.full((L,), -jnp.inf, jnp.float32), unroll=8
            )
            def vmx(j, c):
                return jnp.maximum(c, rows_v[d, r, pl.ds(j * L, L)])

            vmax_s = jnp.max(vmx)
            hit = lane0 & (lab_vec == 1) & (vmax_s < THRESH)
            return jnp.where(hit, THRESH - vmax_s, 0.0)

        def cluster_branch(_):

            @plsc.parallel_loop(
                0, NCH, carry=jnp.zeros((L,), jnp.int32), unroll=4
            )
            def n_splat(j, cnt):
                v_c = rows_v[d, r, pl.ds(j * L, L)]
                m = v_c >= THRESH
                s_c = plsc.cumsum(m.astype(jnp.int32)) + cnt
                gi = j * L + iota
                plsc.store_scatter(posbuf, [s_c - 1], gi, mask=m)
                return cnt + plsc.all_reduce_population_count(m)

            n_s = n_splat[0]

            nch = (n_s + (L - 1)) // L
            si0 = jnp.zeros((L,), jnp.int32)
            mink0 = jnp.full((L,), BIG, jnp.int32)

            @plsc.parallel_loop(0, nch, carry=(si0, si0, si0, mink0))
            def p2_out(j, c):
                si, prevlast, _, mink = c
                pos_c = posbuf[pl.ds(j * L, L)]
                gi = j * L + iota
                validm = gi < n_splat
                prev = jnp.where(lane0, prevlast, _vgather(pos_c, shl))
                gap = pos_c - prev
                newflag = (gap > GAP) & (gi > 0) & validm
                startval = jnp.where(newflag | (gi == 0), gi, 0)
                incl_loc = plsc.cummax(startval)
                excl = jnp.where(
                    lane0, si, jnp.maximum(_vgather(incl_loc, shl), si)
                )
                key = jnp.where(newflag, (gi - excl) * 1024 + prev, BIG)
                mink = jnp.minimum(mink, key)
                si = jnp.maximum(si, _splat_last(incl_loc))
                return si, _splat_last(pos_c), pos_c, mink

            si_f, _, lastp, mink_vec = p2_out

            lanes_last = jnp.bitwise_and(n_s - 1, L - 1)
            lp_s = _vgather(lastp, jnp.full((L,), lanes_last, jnp.int32))[0]
            mk = jnp.min(mink_vec)
            keyf = (n_s - si_f[0]) * 1024 + lp_s
            mk = jnp.minimum(mk, jnp.where(n_s > 0, keyf, BIG))
            tstar = jnp.bitwise_and(mk, 1023)

            vat = plsc.load_gather(
                rows_v,
                [
                    jnp.full((L,), d, jnp.int32),
                    jnp.full((L,), r, jnp.int32),
                    jnp.full((L,), tstar, jnp.int32),
                ],
            )
            return jnp.where(lane0 & (n_splat > 0), vat - THRESH, 0.0)

        contrib = lax.cond(lab_s == 0, cluster_branch, miss_branch, 0)
        return acc + contrib

    def _block_copy(b, d):
        return pltpu.make_async_copy(
            v_hbm.at[pl.ds(wstart + b * RB, RB)], rows_v.at[d], sems.at[d]
        )

    def blk_body(b, acc):
        d = jnp.bitwise_and(b, 1)
        _block_copy(b, d).wait()

        @pl.when(b + 1 < NBLK)
        def _():
            _block_copy(b + 1, 1 - d).start()

        rbase = b * RB

        def rloop(r, a):
            return row_body(r, rbase, d, a)

        return lax.fori_loop(0, RB, rloop, acc)

    _block_copy(0, 0).start()
    acc = lax.fori_loop(0, NBLK, blk_body, jnp.zeros((L,), jnp.float32))
    accbuf[...] = acc
    pltpu.sync_copy(accbuf, out_hbm.at[wid])


@functools.partial(
    pl.kernel,
    out_type=jax.ShapeDtypeStruct((NW, L), jnp.float32),
    compiler_params=pltpu.CompilerParams(needs_layout_passes=False),
    mesh=plsc.VectorSubcoreMesh(core_axis_name="c", subcore_axis_name="s"),
    scratch_types=[
        pltpu.VMEM((2, RB, T), jnp.float32),
        pltpu.VMEM((T,), jnp.int32),
        pltpu.VMEM((ROWS_PER_W,), jnp.int32),
        pltpu.VMEM((L,), jnp.float32),
        pltpu.SemaphoreType.DMA((2,)),
    ],
)
def _stca_loss_sc(v_hbm, lab_hbm, out_hbm, rows_v, posbuf, labels_v, accbuf, sems):
    _sc_body(v_hbm, lab_hbm, out_hbm, rows_v, posbuf, labels_v, accbuf, sems)


def _tc_block(v_ref, lab_ref, out_ref):
    v = v_ref[...]
    lab = lab_ref[0, 0, :]
    spike_b = v >= THRESH
    ru = lax.broadcasted_iota(jnp.int32, (T, T), 0)
    ct = lax.broadcasted_iota(jnp.int32, (T, T), 1)
    tri = (ru <= ct).astype(jnp.bfloat16)
    s = lax.dot_general(
        spike_b.astype(jnp.bfloat16),
        tri,
        (((1,), (0,)), ((), ())),
        preferred_element_type=jnp.float32,
    )
    z = jnp.zeros((RTC, 6), jnp.float32)
    sm1 = jnp.concatenate([z[:, :1], s[:, :-1]], axis=1)
    sm6 = jnp.concatenate([z, s[:, :-6]], axis=1)
    sp5 = jnp.concatenate([s[:, 5:], jnp.broadcast_to(s[:, -1:], (RTC, 5))], axis=1)
    start = spike_b & (sm1 == sm6)
    end = spike_b & (sp5 == s)
    S0 = jnp.where(start, s, 0.0)
    k = 1
    while k < T:
        S0 = jnp.maximum(
            S0,
            jnp.concatenate([jnp.zeros((RTC, k), jnp.float32), S0[:, :-k]], axis=1),
        )
        k *= 2
    size = (s - S0).astype(jnp.int32) + 1
    tt = lax.broadcasted_iota(jnp.int32, (RTC, T), 1)
    key = jnp.where(end, size * 1024 + tt, BIG)
    mk = jnp.min(key, axis=1)
    tstar = jnp.bitwise_and(mk, 1023)
    vat = jnp.sum(jnp.where(tt == tstar[:, None], v, 0.0), axis=1)
    vmax = jnp.max(v, axis=1)
    has = s[:, -1] > 0.0
    contrib = jnp.where((lab == 0) & has, vat - THRESH, 0.0)
    contrib = contrib + jnp.where((lab == 1) & (~has), THRESH - vmax, 0.0)
    out_ref[0, 0, :] = jnp.broadcast_to(jnp.sum(contrib), (128,))


_NB_TC = SPLIT // RTC

_stca_loss_tc = pl.pallas_call(
    _tc_block,
    grid=(_NB_TC,),
    in_specs=[
        pl.BlockSpec((RTC, T), lambda i: (i, 0)),
        pl.BlockSpec((1, 1, RTC), lambda i: (i, 0, 0)),
    ],
    out_specs=pl.BlockSpec((1, 1, 128), lambda i: (i, 0, 0)),
    out_shape=jax.ShapeDtypeStruct((_NB_TC, 1, 128), jnp.float32),
)


def kernel(vmem, labels):
    B, N, Tdim = vmem.shape
    v2 = vmem.reshape(B * N, Tdim)
    lab = labels.reshape(B * N).astype(jnp.int32)
    lab3 = lab.reshape(ROWS // RTC, 1, RTC)
    tc_partials = _stca_loss_tc(v2, lab3)
    sc_partials = _stca_loss_sc(v2, lab)
    return jnp.sum(sc_partials) + jnp.sum(tc_partials[:, 0, 0])


# speedup vs baseline: 1.3698x; 1.3698x over previous
import functools

import jax
import jax.numpy as jnp
from jax import lax
from jax.experimental import pallas as pl
from jax.experimental.pallas import tpu as pltpu
from jax.experimental.pallas import tpu_sc as plsc

THRESH = 1.0
GAP = 5
T = 512
L = 16
NC = 2
NS = 16
NW = NC * NS
ROWS = 32768
SPLIT = 11264
ROWS_PER_W = (ROWS - SPLIT) // NW
RB = 32
NBLK = ROWS_PER_W // RB
NCH = T // L
BIG = 1 << 22
RTC = 256

_GDN = lax.GatherDimensionNumbers(
    offset_dims=(), collapsed_slice_dims=(0,), start_index_map=(0,)
)


def _vgather(x, idx):
    return lax.gather(
        x,
        idx[:, None],
        _GDN,
        slice_sizes=(1,),
        mode=lax.GatherScatterMode.PROMISE_IN_BOUNDS,
    )


def _splat_last(x):
    return _vgather(x, jnp.full((L,), L - 1, jnp.int32))


def _sc_body(v_hbm, lab_hbm, out_hbm, rows_v, posbuf, labels_v, accbuf, sems):
    wid = lax.axis_index("s") * NC + lax.axis_index("c")
    wstart = SPLIT + wid * ROWS_PER_W
    iota = lax.iota(jnp.int32, L)
    lane0 = iota == 0
    shl = jnp.maximum(iota - 1, 0)

    pltpu.sync_copy(lab_hbm.at[pl.ds(wstart, ROWS_PER_W)], labels_v)

    def row_body(r, rbase, d, acc):
        g = rbase + r
        labv = labels_v[pl.ds((g // L) * L, L)]
        lab_vec = _vgather(labv, jnp.full((L,), g % L, jnp.int32))
        lab_s = lab_vec[0]

        def miss_branch(_):
            @plsc.parallel_loop(
                0, NCH, carry=jnp.full((L,), -jnp.inf, jnp.float32), unroll=8
            )
            def vmx(j, c):
                return jnp.maximum(c, rows_v[d, r, pl.ds(j * L, L)])

            vmax_s = jnp.max(vmx)
            hit = lane0 & (lab_vec == 1) & (vmax_s < THRESH)
            return jnp.where(hit, THRESH - vmax_s, 0.0)

        def cluster_branch(_):

            @plsc.parallel_loop(
                0, NCH, carry=jnp.zeros((L,), jnp.int32), unroll=4
            )
            def n_splat(j, cnt):
                v_c = rows_v[d, r, pl.ds(j * L, L)]
                m = v_c >= THRESH
                s_c = plsc.cumsum(m.astype(jnp.int32)) + cnt
                gi = j * L + iota
                plsc.store_scatter(posbuf, [s_c - 1], gi, mask=m)
                return cnt + plsc.all_reduce_population_count(m)

            n_s = n_splat[0]

            nch = (n_s + (L - 1)) // L
            si0 = jnp.zeros((L,), jnp.int32)
            mink0 = jnp.full((L,), BIG, jnp.int32)

            @plsc.parallel_loop(0, nch, carry=(si0, si0, si0, mink0))
            def p2_out(j, c):
                si, prevlast, _, mink = c
                pos_c = posbuf[pl.ds(j * L, L)]
                gi = j * L + iota
                validm = gi < n_splat
                prev = jnp.where(lane0, prevlast, _vgather(pos_c, shl))
                gap = pos_c - prev
                newflag = (gap > GAP) & (gi > 0) & validm
                startval = jnp.where(newflag | (gi == 0), gi, 0)
                incl_loc = plsc.cummax(startval)
                excl = jnp.where(
                    lane0, si, jnp.maximum(_vgather(incl_loc, shl), si)
                )
                key = jnp.where(newflag, (gi - excl) * 1024 + prev, BIG)
                mink = jnp.minimum(mink, key)
                si = jnp.maximum(si, _splat_last(incl_loc))
                return si, _splat_last(pos_c), pos_c, mink

            si_f, _, lastp, mink_vec = p2_out

            lanes_last = jnp.bitwise_and(n_s - 1, L - 1)
            lp_s = _vgather(lastp, jnp.full((L,), lanes_last, jnp.int32))[0]
            mk = jnp.min(mink_vec)
            keyf = (n_s - si_f[0]) * 1024 + lp_s
            mk = jnp.minimum(mk, jnp.where(n_s > 0, keyf, BIG))
            tstar = jnp.bitwise_and(mk, 1023)

            vat = plsc.load_gather(
                rows_v,
                [
                    jnp.full((L,), d, jnp.int32),
                    jnp.full((L,), r, jnp.int32),
                    jnp.full((L,), tstar, jnp.int32),
                ],
            )
            return jnp.where(lane0 & (n_splat > 0), vat - THRESH, 0.0)

        contrib = lax.cond(lab_s == 0, cluster_branch, miss_branch, 0)
        return acc + contrib

    def _block_copy(b, d):
        return pltpu.make_async_copy(
            v_hbm.at[pl.ds(wstart + b * RB, RB)], rows_v.at[d], sems.at[d]
        )

    def blk_body(b, acc):
        d = jnp.bitwise_and(b, 1)
        _block_copy(b, d).wait()

        @pl.when(b + 1 < NBLK)
        def _():
            _block_copy(b + 1, 1 - d).start()

        rbase = b * RB

        def rloop(r, a):
            return row_body(r, rbase, d, a)

        return lax.fori_loop(0, RB, rloop, acc)

    _block_copy(0, 0).start()
    acc = lax.fori_loop(0, NBLK, blk_body, jnp.zeros((L,), jnp.float32))
    accbuf[...] = acc
    pltpu.sync_copy(accbuf, out_hbm.at[wid])


@functools.partial(
    pl.kernel,
    out_type=jax.ShapeDtypeStruct((NW, L), jnp.float32),
    compiler_params=pltpu.CompilerParams(needs_layout_passes=False),
    mesh=plsc.VectorSubcoreMesh(core_axis_name="c", subcore_axis_name="s"),
    scratch_types=[
        pltpu.VMEM((2, RB, T), jnp.float32),
        pltpu.VMEM((T,), jnp.int32),
        pltpu.VMEM((ROWS_PER_W,), jnp.int32),
        pltpu.VMEM((L,), jnp.float32),
        pltpu.SemaphoreType.DMA((2,)),
    ],
)
def _stca_loss_sc(v_hbm, lab_hbm, out_hbm, rows_v, posbuf, labels_v, accbuf, sems):
    _sc_body(v_hbm, lab_hbm, out_hbm, rows_v, posbuf, labels_v, accbuf, sems)


def _tc_block(v_ref, lab_ref, tri_ref, out_ref):
    v = v_ref[...]
    lab = lab_ref[0, 0, :]
    spike_b = v >= THRESH
    s = lax.dot_general(
        spike_b.astype(jnp.bfloat16),
        tri_ref[...],
        (((1,), (0,)), ((), ())),
        preferred_element_type=jnp.float32,
    )
    z = jnp.zeros((RTC, 6), jnp.float32)
    sm1 = jnp.concatenate([z[:, :1], s[:, :-1]], axis=1)
    sm6 = jnp.concatenate([z, s[:, :-6]], axis=1)
    sp5 = jnp.concatenate([s[:, 5:], jnp.broadcast_to(s[:, -1:], (RTC, 5))], axis=1)
    start = spike_b & (sm1 == sm6)
    end = spike_b & (sp5 == s)
    S0 = jnp.where(start, s, 0.0)
    k = 1
    while k < T:
        S0 = jnp.maximum(
            S0,
            jnp.concatenate([jnp.zeros((RTC, k), jnp.float32), S0[:, :-k]], axis=1),
        )
        k *= 2
    size = (s - S0).astype(jnp.int32) + 1
    tt = lax.broadcasted_iota(jnp.int32, (RTC, T), 1)
    key = jnp.where(end, size * 1024 + tt, BIG)
    mk = jnp.min(key, axis=1)
    tstar = jnp.bitwise_and(mk, 1023)
    vat = jnp.sum(jnp.where(tt == tstar[:, None], v, 0.0), axis=1)
    vmax = jnp.max(v, axis=1)
    has = s[:, -1] > 0.0
    contrib = jnp.where((lab == 0) & has, vat - THRESH, 0.0)
    contrib = contrib + jnp.where((lab == 1) & (~has), THRESH - vmax, 0.0)
    out_ref[0, 0, :] = jnp.broadcast_to(jnp.sum(contrib), (128,))


_NB_TC = SPLIT // RTC

_stca_loss_tc = pl.pallas_call(
    _tc_block,
    grid=(_NB_TC,),
    in_specs=[
        pl.BlockSpec((RTC, T), lambda i: (i, 0)),
        pl.BlockSpec((1, 1, RTC), lambda i: (i, 0, 0)),
        pl.BlockSpec((T, T), lambda i: (0, 0)),
    ],
    out_specs=pl.BlockSpec((1, 1, 128), lambda i: (i, 0, 0)),
    out_shape=jax.ShapeDtypeStruct((_NB_TC, 1, 128), jnp.float32),
)


def kernel(vmem, labels):
    B, N, Tdim = vmem.shape
    v2 = vmem.reshape(B * N, Tdim)
    lab = labels.reshape(B * N).astype(jnp.int32)
    lab3 = lab.reshape(ROWS // RTC, 1, RTC)
    ru = lax.broadcasted_iota(jnp.int32, (T, T), 0)
    ct = lax.broadcasted_iota(jnp.int32, (T, T), 1)
    tri = (ru <= ct).astype(jnp.bfloat16)
    tc_partials = _stca_loss_tc(v2, lab3, tri)
    sc_partials = _stca_loss_sc(v2, lab)
    return jnp.sum(sc_partials) + jnp.sum(tc_partials[:, 0, 0])


# speedup vs baseline: 1.3744x; 1.0034x over previous
import functools

import numpy as np

import jax
import jax.numpy as jnp
from jax import lax
from jax.experimental import pallas as pl
from jax.experimental.pallas import tpu as pltpu
from jax.experimental.pallas import tpu_sc as plsc

THRESH = 1.0
GAP = 5
T = 512
L = 16
NC = 2
NS = 16
NW = NC * NS
ROWS = 32768
SPLIT = 11264
ROWS_PER_W = (ROWS - SPLIT) // NW
RB = 32
NBLK = ROWS_PER_W // RB
NCH = T // L
BIG = 1 << 22
RTC = 256

_GDN = lax.GatherDimensionNumbers(
    offset_dims=(), collapsed_slice_dims=(0,), start_index_map=(0,)
)


def _vgather(x, idx):
    return lax.gather(
        x,
        idx[:, None],
        _GDN,
        slice_sizes=(1,),
        mode=lax.GatherScatterMode.PROMISE_IN_BOUNDS,
    )


def _splat_last(x):
    return _vgather(x, jnp.full((L,), L - 1, jnp.int32))


def _sc_body(v_hbm, lab_hbm, out_hbm, rows_v, posbuf, labels_v, accbuf, sems):
    wid = lax.axis_index("s") * NC + lax.axis_index("c")
    wstart = SPLIT + wid * ROWS_PER_W
    iota = lax.iota(jnp.int32, L)
    lane0 = iota == 0
    shl = jnp.maximum(iota - 1, 0)

    pltpu.sync_copy(lab_hbm.at[pl.ds(wstart, ROWS_PER_W)], labels_v)

    def row_body(r, rbase, d, acc):
        g = rbase + r
        labv = labels_v[pl.ds((g // L) * L, L)]
        lab_vec = _vgather(labv, jnp.full((L,), g % L, jnp.int32))
        lab_s = lab_vec[0]

        def miss_branch(_):
            @plsc.parallel_loop(
                0, NCH, carry=jnp.full((L,), -jnp.inf, jnp.float32), unroll=8
            )
            def vmx(j, c):
                return jnp.maximum(c, rows_v[d, r, pl.ds(j * L, L)])

            vmax_s = jnp.max(vmx)
            hit = lane0 & (lab_vec == 1) & (vmax_s < THRESH)
            return jnp.where(hit, THRESH - vmax_s, 0.0)

        def cluster_branch(_):

            @plsc.parallel_loop(
                0, NCH, carry=jnp.zeros((L,), jnp.int32), unroll=4
            )
            def n_splat(j, cnt):
                v_c = rows_v[d, r, pl.ds(j * L, L)]
                m = v_c >= THRESH
                s_c = plsc.cumsum(m.astype(jnp.int32)) + cnt
                gi = j * L + iota
                plsc.store_scatter(posbuf, [s_c - 1], gi, mask=m)
                return cnt + plsc.all_reduce_population_count(m)

            n_s = n_splat[0]

            nch = (n_s + (L - 1)) // L
            si0 = jnp.zeros((L,), jnp.int32)
            mink0 = jnp.full((L,), BIG, jnp.int32)

            @plsc.parallel_loop(0, nch, carry=(si0, si0, si0, mink0))
            def p2_out(j, c):
                si, prevlast, _, mink = c
                pos_c = posbuf[pl.ds(j * L, L)]
                gi = j * L + iota
                validm = gi < n_splat
                prev = jnp.where(lane0, prevlast, _vgather(pos_c, shl))
                gap = pos_c - prev
                newflag = (gap > GAP) & (gi > 0) & validm
                startval = jnp.where(newflag | (gi == 0), gi, 0)
                incl_loc = plsc.cummax(startval)
                excl = jnp.where(
                    lane0, si, jnp.maximum(_vgather(incl_loc, shl), si)
                )
                key = jnp.where(newflag, (gi - excl) * 1024 + prev, BIG)
                mink = jnp.minimum(mink, key)
                si = jnp.maximum(si, _splat_last(incl_loc))
                return si, _splat_last(pos_c), pos_c, mink

            si_f, _, lastp, mink_vec = p2_out

            lanes_last = jnp.bitwise_and(n_s - 1, L - 1)
            lp_s = _vgather(lastp, jnp.full((L,), lanes_last, jnp.int32))[0]
            mk = jnp.min(mink_vec)
            keyf = (n_s - si_f[0]) * 1024 + lp_s
            mk = jnp.minimum(mk, jnp.where(n_s > 0, keyf, BIG))
            tstar = jnp.bitwise_and(mk, 1023)

            vat = plsc.load_gather(
                rows_v,
                [
                    jnp.full((L,), d, jnp.int32),
                    jnp.full((L,), r, jnp.int32),
                    jnp.full((L,), tstar, jnp.int32),
                ],
            )
            return jnp.where(lane0 & (n_splat > 0), vat - THRESH, 0.0)

        contrib = lax.cond(lab_s == 0, cluster_branch, miss_branch, 0)
        return acc + contrib

    def _block_copy(b, d):
        return pltpu.make_async_copy(
            v_hbm.at[pl.ds(wstart + b * RB, RB)], rows_v.at[d], sems.at[d]
        )

    def blk_body(b, acc):
        d = jnp.bitwise_and(b, 1)
        _block_copy(b, d).wait()

        @pl.when(b + 1 < NBLK)
        def _():
            _block_copy(b + 1, 1 - d).start()

        rbase = b * RB

        def rloop(r, a):
            return row_body(r, rbase, d, a)

        return lax.fori_loop(0, RB, rloop, acc)

    _block_copy(0, 0).start()
    acc = lax.fori_loop(0, NBLK, blk_body, jnp.zeros((L,), jnp.float32))
    accbuf[...] = acc
    pltpu.sync_copy(accbuf, out_hbm.at[wid])


@functools.partial(
    pl.kernel,
    out_type=jax.ShapeDtypeStruct((NW, L), jnp.float32),
    compiler_params=pltpu.CompilerParams(needs_layout_passes=False),
    mesh=plsc.VectorSubcoreMesh(core_axis_name="c", subcore_axis_name="s"),
    scratch_types=[
        pltpu.VMEM((2, RB, T), jnp.float32),
        pltpu.VMEM((T,), jnp.int32),
        pltpu.VMEM((ROWS_PER_W,), jnp.int32),
        pltpu.VMEM((L,), jnp.float32),
        pltpu.SemaphoreType.DMA((2,)),
    ],
)
def _stca_loss_sc(v_hbm, lab_hbm, out_hbm, rows_v, posbuf, labels_v, accbuf, sems):
    _sc_body(v_hbm, lab_hbm, out_hbm, rows_v, posbuf, labels_v, accbuf, sems)


def _tc_block(v_ref, lab_ref, tri_ref, out_ref):
    v = v_ref[...]
    lab = lab_ref[0, 0, :]
    spike_b = v >= THRESH
    s = lax.dot_general(
        spike_b.astype(jnp.bfloat16),
        tri_ref[...],
        (((1,), (0,)), ((), ())),
        preferred_element_type=jnp.float32,
    )
    z = jnp.zeros((RTC, 6), jnp.float32)
    sm1 = jnp.concatenate([z[:, :1], s[:, :-1]], axis=1)
    sm6 = jnp.concatenate([z, s[:, :-6]], axis=1)
    sp5 = jnp.concatenate([s[:, 5:], jnp.broadcast_to(s[:, -1:], (RTC, 5))], axis=1)
    start = spike_b & (sm1 == sm6)
    end = spike_b & (sp5 == s)
    S0 = jnp.where(start, s, 0.0)
    k = 1
    while k < T:
        S0 = jnp.maximum(
            S0,
            jnp.concatenate([jnp.zeros((RTC, k), jnp.float32), S0[:, :-k]], axis=1),
        )
        k *= 2
    size = (s - S0).astype(jnp.int32) + 1
    tt = lax.broadcasted_iota(jnp.int32, (RTC, T), 1)
    key = jnp.where(end, size * 1024 + tt, BIG)
    mk = jnp.min(key, axis=1)
    tstar = jnp.bitwise_and(mk, 1023)
    vat = jnp.sum(jnp.where(tt == tstar[:, None], v, 0.0), axis=1)
    vmax = jnp.max(v, axis=1)
    has = s[:, -1] > 0.0
    contrib = jnp.where((lab == 0) & has, vat - THRESH, 0.0)
    contrib = contrib + jnp.where((lab == 1) & (~has), THRESH - vmax, 0.0)
    out_ref[0, 0, :] = jnp.broadcast_to(jnp.sum(contrib), (128,))


_NB_TC = SPLIT // RTC

_TRI = jnp.asarray(np.triu(np.ones((T, T), np.float32)), dtype=jnp.bfloat16)

_stca_loss_tc = pl.pallas_call(
    _tc_block,
    grid=(_NB_TC,),
    in_specs=[
        pl.BlockSpec((RTC, T), lambda i: (i, 0)),
        pl.BlockSpec((1, 1, RTC), lambda i: (i, 0, 0)),
        pl.BlockSpec((T, T), lambda i: (0, 0)),
    ],
    out_specs=pl.BlockSpec((1, 1, 128), lambda i: (i, 0, 0)),
    out_shape=jax.ShapeDtypeStruct((_NB_TC, 1, 128), jnp.float32),
)


def kernel(vmem, labels):
    B, N, Tdim = vmem.shape
    v2 = vmem.reshape(B * N, Tdim)
    lab = labels.reshape(B * N).astype(jnp.int32)
    lab3 = lab.reshape(ROWS // RTC, 1, RTC)
    tc_partials = _stca_loss_tc(v2, lab3, _TRI)
    sc_partials = _stca_loss_sc(v2, lab)
    return jnp.sum(sc_partials) + jnp.sum(tc_partials[:, 0, 0])
